# baseline (speedup 1.0000x reference)
import functools

import jax
import jax.numpy as jnp
from jax import lax
from jax.experimental import pallas as pl
from jax.experimental.pallas import tpu as pltpu
from jax.experimental.pallas import tpu_sc as plsc

B = 16384
N_DOMAINS = 8
L = 16
NC = 2
NS = 16
NW = NC * NS
CHUNK = B // NW
NVEC = CHUNK // L


def _softplus_table(rw_vec, tmp_ref):
    lane = lax.iota(jnp.int32, L)
    valid = lane < N_DOMAINS
    x = jnp.where(valid, rw_vec, 0.0)
    a = 1.0 + jnp.exp(x)
    y = jnp.maximum(x, 0.0)
    for _ in range(5):
        y = y + (a * jnp.exp(-y) - 1.0)
    w = jnp.where(valid, y, 0.0)
    s = w
    tmp_ref[...] = s
    for shift in (1, 2, 4):
        s = s + plsc.load_gather(tmp_ref, [lane ^ shift])
        if shift != 4:
            tmp_ref[...] = s
    mean = jnp.where(valid, s, 1.0) * (1.0 / N_DOMAINS)
    return w / mean


def _make_sc_call():
    mesh = plsc.VectorSubcoreMesh(core_axis_name="c", subcore_axis_name="s")

    @functools.partial(
        pl.kernel,
        mesh=mesh,
        out_type=jax.ShapeDtypeStruct((B,), jnp.float32),
        compiler_params=pltpu.CompilerParams(
            needs_layout_passes=False,
            skip_device_barrier=True,
            disable_bounds_checks=True,
            disable_semaphore_checks=True,
        ),
        scratch_types=[
            pltpu.VMEM((L,), jnp.float32),
            pltpu.VMEM((L,), jnp.float32),
            pltpu.VMEM((CHUNK,), jnp.int32),
            pltpu.VMEM((CHUNK,), jnp.float32),
            pltpu.VMEM((CHUNK,), jnp.float32),
            pltpu.SemaphoreType.DMA,
            pltpu.SemaphoreType.DMA,
            pltpu.SemaphoreType.DMA,
        ],
    )
    def sc_kernel(idx_hbm, rew_hbm, rw_hbm, out_hbm, tab_v, tmp_v,
                  idx_v, rew_v, out_v, sem_t, sem_i, sem_r):
        wid = lax.axis_index("s") * NC + lax.axis_index("c")
        base = wid * CHUNK
        cp_t = pltpu.async_copy(rw_hbm, tab_v.at[pl.ds(0, N_DOMAINS)], sem_t)
        cp_i = pltpu.async_copy(idx_hbm.at[pl.ds(base, CHUNK)], idx_v, sem_i)
        cp_r = pltpu.async_copy(rew_hbm.at[pl.ds(base, CHUNK)], rew_v, sem_r)
        cp_t.wait()
        tab_v[...] = _softplus_table(tab_v[...], tmp_v)
        cp_i.wait()
        cp_r.wait()

        def body(i, carry):
            sl = pl.ds(i * L, L)
            g = plsc.load_gather(tab_v, [idx_v[sl]])
            out_v[sl] = rew_v[sl] * g
            return carry

        lax.fori_loop(0, NVEC, body, 0)
        pltpu.sync_copy(out_v, out_hbm.at[pl.ds(base, CHUNK)])

    return sc_kernel


_sc_call = _make_sc_call()


def kernel(domain_indices, rewards, raw_weights):
    idx = domain_indices.astype(jnp.int32)
    rew = rewards.astype(jnp.float32)
    out = _sc_call(idx, rew, raw_weights.astype(jnp.float32))
    if rewards.ndim == 1:
        return out
    return out.reshape(rewards.shape)


# speedup vs baseline: 1.0573x; 1.0573x over previous
import functools

import jax
import jax.numpy as jnp
from jax import lax
from jax.experimental import pallas as pl
from jax.experimental.pallas import tpu as pltpu
from jax.experimental.pallas import tpu_sc as plsc

B = 16384
N_DOMAINS = 8
L = 16
NC = 1
NS = 16
NW = NC * NS
CHUNK = B // NW
NVEC = CHUNK // L


def _softplus_table(rw_vec, tmp_ref):
    lane = lax.iota(jnp.int32, L)
    valid = lane < N_DOMAINS
    x = jnp.where(valid, rw_vec, 0.0)
    a = 1.0 + jnp.exp(x)
    y = jnp.maximum(x, 0.0)
    for _ in range(5):
        y = y + (a * jnp.exp(-y) - 1.0)
    w = jnp.where(valid, y, 0.0)
    s = w
    tmp_ref[...] = s
    for shift in (1, 2, 4):
        s = s + plsc.load_gather(tmp_ref, [lane ^ shift])
        if shift != 4:
            tmp_ref[...] = s
    mean = jnp.where(valid, s, 1.0) * (1.0 / N_DOMAINS)
    return w / mean


def _make_sc_call():
    mesh = plsc.VectorSubcoreMesh(core_axis_name="c", subcore_axis_name="s", num_cores=1)

    @functools.partial(
        pl.kernel,
        mesh=mesh,
        out_type=jax.ShapeDtypeStruct((B,), jnp.float32),
        compiler_params=pltpu.CompilerParams(
            needs_layout_passes=False,
            skip_device_barrier=True,
            disable_bounds_checks=True,
            disable_semaphore_checks=True,
        ),
        scratch_types=[
            pltpu.VMEM((L,), jnp.float32),
            pltpu.VMEM((L,), jnp.float32),
            pltpu.VMEM((CHUNK,), jnp.int32),
            pltpu.VMEM((CHUNK,), jnp.float32),
            pltpu.VMEM((CHUNK,), jnp.float32),
            pltpu.SemaphoreType.DMA,
            pltpu.SemaphoreType.DMA,
            pltpu.SemaphoreType.DMA,
        ],
    )
    def sc_kernel(idx_hbm, rew_hbm, rw_hbm, out_hbm, tab_v, tmp_v,
                  idx_v, rew_v, out_v, sem_t, sem_i, sem_r):
        wid = lax.axis_index("s") * NC + lax.axis_index("c")
        base = wid * CHUNK
        cp_t = pltpu.async_copy(rw_hbm, tab_v.at[pl.ds(0, N_DOMAINS)], sem_t)
        cp_i = pltpu.async_copy(idx_hbm.at[pl.ds(base, CHUNK)], idx_v, sem_i)
        cp_r = pltpu.async_copy(rew_hbm.at[pl.ds(base, CHUNK)], rew_v, sem_r)
        cp_t.wait()
        tab_v[...] = _softplus_table(tab_v[...], tmp_v)
        cp_i.wait()
        cp_r.wait()

        def body(i, carry):
            sl = pl.ds(i * L, L)
            g = plsc.load_gather(tab_v, [idx_v[sl]])
            out_v[sl] = rew_v[sl] * g
            return carry

        lax.fori_loop(0, NVEC, body, 0)
        pltpu.sync_copy(out_v, out_hbm.at[pl.ds(base, CHUNK)])

    return sc_kernel


_sc_call = _make_sc_call()


def kernel(domain_indices, rewards, raw_weights):
    idx = domain_indices.astype(jnp.int32)
    rew = rewards.astype(jnp.float32)
    out = _sc_call(idx, rew, raw_weights.astype(jnp.float32))
    if rewards.ndim == 1:
        return out
    return out.reshape(rewards.shape)


# speedup vs baseline: 1.0638x; 1.0062x over previous
import functools

import jax
import jax.numpy as jnp
from jax import lax
from jax.experimental import pallas as pl
from jax.experimental.pallas import tpu as pltpu
from jax.experimental.pallas import tpu_sc as plsc

B = 16384
N_DOMAINS = 8
L = 16
NC = 1
NS = 16
NW = NC * NS
CHUNK = B // NW
NVEC = CHUNK // L


def _softplus_table(rw_vec, tmp_ref):
    lane = lax.iota(jnp.int32, L)
    valid = lane < N_DOMAINS
    x = jnp.where(valid, rw_vec, 0.0)
    a = 1.0 + jnp.exp(x)
    y = jnp.maximum(x, 0.0)
    for _ in range(5):
        y = y + (a * jnp.exp(-y) - 1.0)
    w = jnp.where(valid, y, 0.0)
    s = w
    for shift in (1, 2, 4):
        tmp_ref[...] = s
        s = s + plsc.load_gather(tmp_ref, [lane ^ shift])
    mean = jnp.where(valid, s, 1.0) * (1.0 / N_DOMAINS)
    return w / mean


def _make_sc_call():
    mesh = plsc.VectorSubcoreMesh(core_axis_name="c", subcore_axis_name="s", num_cores=1)

    @functools.partial(
        pl.kernel,
        mesh=mesh,
        out_type=jax.ShapeDtypeStruct((B,), jnp.float32),
        compiler_params=pltpu.CompilerParams(
            needs_layout_passes=False,
            skip_device_barrier=True,
            disable_bounds_checks=True,
            disable_semaphore_checks=True,
        ),
        scratch_types=[
            pltpu.VMEM((L,), jnp.float32),
            pltpu.VMEM((L,), jnp.float32),
            pltpu.VMEM((CHUNK,), jnp.int32),
            pltpu.VMEM((CHUNK,), jnp.float32),
            pltpu.SemaphoreType.DMA,
        ],
    )
    def sc_kernel(idx_hbm, rew_hbm, rw_hbm, out_hbm, tab_v, tmp_v,
                  idx_v, rew_v, sem):
        wid = lax.axis_index("s") * NC + lax.axis_index("c")
        base = wid * CHUNK
        cp_t = pltpu.async_copy(rw_hbm, tab_v.at[pl.ds(0, N_DOMAINS)], sem)
        cp_i = pltpu.async_copy(idx_hbm.at[pl.ds(base, CHUNK)], idx_v, sem)
        cp_r = pltpu.async_copy(rew_hbm.at[pl.ds(base, CHUNK)], rew_v, sem)
        cp_t.wait()
        cp_i.wait()
        cp_r.wait()
        tab_v[...] = _softplus_table(tab_v[...], tmp_v)

        def body(i, carry):
            sl = pl.ds(i * L, L)
            g = plsc.load_gather(tab_v, [idx_v[sl]])
            rew_v[sl] = rew_v[sl] * g
            return carry

        lax.fori_loop(0, NVEC, body, 0)
        pltpu.sync_copy(rew_v, out_hbm.at[pl.ds(base, CHUNK)])

    return sc_kernel


_sc_call = _make_sc_call()


def kernel(domain_indices, rewards, raw_weights):
    idx = domain_indices.astype(jnp.int32)
    rew = rewards.astype(jnp.float32)
    out = _sc_call(idx, rew, raw_weights.astype(jnp.float32))
    if rewards.ndim == 1:
        return out
    return out.reshape(rewards.shape)


# speedup vs baseline: 1.0662x; 1.0022x over previous
import functools

import jax
import jax.numpy as jnp
from jax import lax
from jax.experimental import pallas as pl
from jax.experimental.pallas import tpu as pltpu
from jax.experimental.pallas import tpu_sc as plsc

B = 16384
N_DOMAINS = 8
L = 16
NC = 1
NS = 16
NW = NC * NS
CHUNK = B // NW
NVEC = CHUNK // L


def _softplus_table(rw_vec, tmp_ref):
    lane = lax.iota(jnp.int32, L)
    valid = lane < N_DOMAINS
    x = jnp.where(valid, rw_vec, 0.0)
    a = 1.0 + jnp.exp(x)
    y = jnp.maximum(x, 0.0)
    for _ in range(5):
        y = y + (a * jnp.exp(-y) - 1.0)
    w = jnp.where(valid, y, 0.0)
    s = w
    for shift in (1, 2, 4):
        tmp_ref[...] = s
        s = s + plsc.load_gather(tmp_ref, [lane ^ shift])
    mean = jnp.where(valid, s, 1.0) * (1.0 / N_DOMAINS)
    return w / mean


def _make_sc_call():
    mesh = plsc.VectorSubcoreMesh(core_axis_name="c", subcore_axis_name="s", num_cores=1)

    @functools.partial(
        pl.kernel,
        mesh=mesh,
        out_type=jax.ShapeDtypeStruct((B,), jnp.float32),
        compiler_params=pltpu.CompilerParams(
            needs_layout_passes=False,
            skip_device_barrier=True,
            disable_bounds_checks=True,
            disable_semaphore_checks=True,
        ),
        scratch_types=[
            pltpu.VMEM((L,), jnp.float32),
            pltpu.VMEM((L,), jnp.float32),
            pltpu.VMEM((CHUNK,), jnp.int32),
            pltpu.VMEM((CHUNK,), jnp.float32),
            pltpu.SemaphoreType.DMA,
        ],
    )
    def sc_kernel(idx_hbm, rew_hbm, rw_hbm, out_hbm, tab_v, tmp_v,
                  idx_v, rew_v, sem):
        wid = lax.axis_index("s") * NC + lax.axis_index("c")
        base = wid * CHUNK
        cp_t = pltpu.async_copy(rw_hbm, tab_v.at[pl.ds(0, N_DOMAINS)], sem)
        cp_i = pltpu.async_copy(idx_hbm.at[pl.ds(base, CHUNK)], idx_v, sem)
        cp_r = pltpu.async_copy(rew_hbm.at[pl.ds(base, CHUNK)], rew_v, sem)
        cp_t.wait()
        cp_i.wait()
        cp_r.wait()
        tab_v[...] = _softplus_table(tab_v[...], tmp_v)

        @plsc.parallel_loop(0, CHUNK, step=L, unroll=4)
        def body(i):
            sl = pl.ds(i, L)
            g = plsc.load_gather(tab_v, [idx_v[sl]])
            rew_v[sl] = rew_v[sl] * g
        pltpu.sync_copy(rew_v, out_hbm.at[pl.ds(base, CHUNK)])

    return sc_kernel


_sc_call = _make_sc_call()


def kernel(domain_indices, rewards, raw_weights):
    idx = domain_indices.astype(jnp.int32)
    rew = rewards.astype(jnp.float32)
    out = _sc_call(idx, rew, raw_weights.astype(jnp.float32))
    if rewards.ndim == 1:
        return out
    return out.reshape(rewards.shape)


# speedup vs baseline: 1.0724x; 1.0058x over previous
import functools

import jax
import jax.numpy as jnp
from jax import lax
from jax.experimental import pallas as pl
from jax.experimental.pallas import tpu as pltpu
from jax.experimental.pallas import tpu_sc as plsc

B = 16384
N_DOMAINS = 8
L = 16
NC = 1
NS = 16
NW = NC * NS
CHUNK = B // NW
NVEC = CHUNK // L


def _softplus_table(rw_vec, tmp_ref):
    lane = lax.iota(jnp.int32, L)
    valid = lane < N_DOMAINS
    x = jnp.where(valid, rw_vec, 0.0)
    a = 1.0 + jnp.exp(x)
    y = jnp.maximum(x, 0.0)
    for _ in range(5):
        y = y + (a * jnp.exp(-y) - 1.0)
    w = jnp.where(valid, y, 0.0)
    s = w
    for shift in (1, 2, 4):
        tmp_ref[...] = s
        s = s + plsc.load_gather(tmp_ref, [lane ^ shift])
    mean = jnp.where(valid, s, 1.0) * (1.0 / N_DOMAINS)
    return w / mean


def _make_sc_call():
    mesh = plsc.VectorSubcoreMesh(core_axis_name="c", subcore_axis_name="s", num_cores=1)

    @functools.partial(
        pl.kernel,
        mesh=mesh,
        out_type=jax.ShapeDtypeStruct((B,), jnp.float32),
        compiler_params=pltpu.CompilerParams(
            needs_layout_passes=False,
            skip_device_barrier=True,
            disable_bounds_checks=True,
            disable_semaphore_checks=True,
        ),
        scratch_types=[
            pltpu.VMEM((L,), jnp.float32),
            pltpu.VMEM((L,), jnp.float32),
            pltpu.VMEM((CHUNK,), jnp.int32),
            pltpu.VMEM((CHUNK,), jnp.float32),
            pltpu.SemaphoreType.DMA,
        ],
    )
    def sc_kernel(idx_hbm, rew_hbm, rw_hbm, out_hbm, tab_v, tmp_v,
                  idx_v, rew_v, sem):
        wid = lax.axis_index("s") * NC + lax.axis_index("c")
        base = wid * CHUNK
        cp_t = pltpu.async_copy(rw_hbm, tab_v.at[pl.ds(0, N_DOMAINS)], sem)
        cp_i = pltpu.async_copy(idx_hbm.at[pl.ds(base, CHUNK)], idx_v, sem)
        cp_r = pltpu.async_copy(rew_hbm.at[pl.ds(base, CHUNK)], rew_v, sem)
        cp_t.wait()
        tab_v[...] = _softplus_table(tab_v[...], tmp_v)
        cp_i.wait()
        cp_r.wait()

        @plsc.parallel_loop(0, CHUNK, step=L, unroll=8)
        def body(i):
            sl = pl.ds(i, L)
            g = plsc.load_gather(tab_v, [idx_v[sl]])
            rew_v[sl] = rew_v[sl] * g
        pltpu.sync_copy(rew_v, out_hbm.at[pl.ds(base, CHUNK)])

    return sc_kernel


_sc_call = _make_sc_call()


def kernel(domain_indices, rewards, raw_weights):
    idx = domain_indices.astype(jnp.int32)
    rew = rewards.astype(jnp.float32)
    out = _sc_call(idx, rew, raw_weights.astype(jnp.float32))
    if rewards.ndim == 1:
        return out
    return out.reshape(rewards.shape)
